# baseline (speedup 1.0000x reference)
import functools

import jax
import jax.numpy as jnp
from jax import lax
from jax.experimental import pallas as pl
from jax.experimental.pallas import tpu as pltpu
from jax.experimental.pallas import tpu_sc as plsc

_FEAT = 64
_BATCH = 16384
_ALPHA = 1.0
_EPS = 1e-12

_NC = 1
_NS = 16
_NW = _NC * _NS
_PASSES = 2
_BPW = _BATCH // (_NW * _PASSES)
_GCHUNK = 128
_NCHUNK = _BPW // _GCHUNK
_GROUPS = _BPW // 16


def _rsqrt16(x):
    y = lax.bitcast_convert_type(x, jnp.int32)
    y = jnp.int32(0x5F3759DF) - (y >> 1)
    r = lax.bitcast_convert_type(y, jnp.float32)
    for _ in range(3):
        r = r * (1.5 - 0.5 * x * r * r)
    return r


def _body(feats_hbm, labels_hbm, centers_hbm, out_hbm, idx_v, f_v, c_v,
          acc_v, sem):
    wid = lax.axis_index("s")

    iota16 = lax.iota(jnp.int32, 16)
    zero16 = jnp.zeros((16,), jnp.float32)
    eps2 = jnp.float32(_EPS * _EPS)

    loss_total = zero16
    for p in range(_PASSES):
        chunk = wid * _PASSES + p
        base = chunk * _BPW

        pltpu.sync_copy(labels_hbm.at[chunk], idx_v)
        gathers = [
            pltpu.async_copy(
                centers_hbm.at[idx_v.at[k]],
                c_v.at[pl.ds(k * _GCHUNK, _GCHUNK)],
                sem,
            )
            for k in range(_NCHUNK)
        ]
        pltpu.sync_copy(feats_hbm.at[pl.ds(base, _BPW)], f_v)
        for g in gathers:
            g.wait()

        def group_body(g, loss_acc):
            rows = g * 16 + iota16

            def col_body(j, carry):
                sf, sc, sfc = carry
                cols = jnp.full((16,), j, dtype=jnp.int32)
                fv = plsc.load_gather(f_v, [rows, cols])
                cv = plsc.load_gather(c_v, [rows, cols])
                return sf + fv * fv, sc + cv * cv, sfc + fv * cv

            sf, sc, sfc = lax.fori_loop(
                0, _FEAT, col_body, (zero16, zero16, zero16))

            mf = jnp.maximum(sf, eps2)
            mc = jnp.maximum(sc, eps2)
            p_ = jnp.maximum(mf * mc, jnp.float32(1e-34))
            loss16 = sf / mf + sc / mc - 2.0 * (sfc * _rsqrt16(p_))
            return loss_acc + loss16

        loss_total = lax.fori_loop(0, _GROUPS, group_body, loss_total)

    acc_v[...] = loss_total
    pltpu.sync_copy(acc_v, out_hbm.at[wid])


@jax.jit
def kernel(feats, labels, centers):
    lab = labels.astype(jnp.int32).reshape(_NW * _PASSES, _NCHUNK, _GCHUNK)
    mesh = plsc.VectorSubcoreMesh(
        core_axis_name="c", subcore_axis_name="s", num_cores=_NC)
    run = functools.partial(
        pl.kernel,
        mesh=mesh,
        compiler_params=pltpu.CompilerParams(
            needs_layout_passes=False, use_tc_tiling_on_sc=False),
        out_type=jax.ShapeDtypeStruct((_NW, 16), jnp.float32),
        scratch_types=[
            pltpu.VMEM((_NCHUNK, _GCHUNK), jnp.int32),
            pltpu.VMEM((_BPW, _FEAT), jnp.float32),
            pltpu.VMEM((_BPW, _FEAT), jnp.float32),
            pltpu.VMEM((16,), jnp.float32),
            pltpu.SemaphoreType.DMA,
        ],
    )(_body)
    partial_losses = run(feats, lab, centers)
    return _ALPHA * (jnp.sum(partial_losses) / _BATCH)


# speedup vs baseline: 1.0548x; 1.0548x over previous
import functools

import jax
import jax.numpy as jnp
from jax import lax
from jax.experimental import pallas as pl
from jax.experimental.pallas import tpu as pltpu
from jax.experimental.pallas import tpu_sc as plsc

_FEAT = 64
_BATCH = 16384
_ALPHA = 1.0
_EPS = 1e-12

_NC = 2
_NS = 16
_NW = _NC * _NS
_BPW = _BATCH // _NW
_GCHUNK = 128
_NCHUNK = _BPW // _GCHUNK
_GROUPS = _BPW // 16


def _rsqrt16(x):
    y = lax.bitcast_convert_type(x, jnp.int32)
    y = jnp.int32(0x5F3759DF) - (y >> 1)
    r = lax.bitcast_convert_type(y, jnp.float32)
    for _ in range(3):
        r = r * (1.5 - 0.5 * x * r * r)
    return r


def _body(feats_hbm, labels_hbm, centers_hbm, out_hbm, idx_v, f_v, c_v,
          acc_v, sem):
    wid = lax.axis_index("s") * _NC + lax.axis_index("c")
    base = wid * _BPW

    pltpu.sync_copy(labels_hbm.at[wid], idx_v)
    gathers = [
        pltpu.async_copy(
            centers_hbm.at[idx_v.at[k]],
            c_v.at[pl.ds(k * _GCHUNK, _GCHUNK)],
            sem,
        )
        for k in range(_NCHUNK)
    ]
    pltpu.sync_copy(feats_hbm.at[pl.ds(base, _BPW)], f_v)
    for g in gathers:
        g.wait()

    iota16 = lax.iota(jnp.int32, 16)
    zero16 = jnp.zeros((16,), jnp.float32)
    eps2 = jnp.float32(_EPS * _EPS)

    def group_body(g, loss_acc):
        rows = g * 16 + iota16

        def col_body(j, carry):
            sf, sc, sfc = carry
            cols = jnp.full((16,), j, dtype=jnp.int32)
            fv = plsc.load_gather(f_v, [rows, cols])
            cv = plsc.load_gather(c_v, [rows, cols])
            return sf + fv * fv, sc + cv * cv, sfc + fv * cv

        sf, sc, sfc = lax.fori_loop(
            0, _FEAT, col_body, (zero16, zero16, zero16))

        mf = jnp.maximum(sf, eps2)
        mc = jnp.maximum(sc, eps2)
        p = jnp.maximum(mf * mc, jnp.float32(1e-34))
        loss16 = sf / mf + sc / mc - 2.0 * (sfc * _rsqrt16(p))
        return loss_acc + loss16

    acc_v[...] = lax.fori_loop(0, _GROUPS, group_body, zero16)
    pltpu.sync_copy(acc_v, out_hbm.at[wid])


@jax.jit
def kernel(feats, labels, centers):
    lab = labels.astype(jnp.int32).reshape(_NW, _NCHUNK, _GCHUNK)
    mesh = plsc.VectorSubcoreMesh(core_axis_name="c", subcore_axis_name="s")
    run = functools.partial(
        pl.kernel,
        mesh=mesh,
        compiler_params=pltpu.CompilerParams(
            needs_layout_passes=False, use_tc_tiling_on_sc=False),
        out_type=jax.ShapeDtypeStruct((_NW, 16), jnp.float32),
        scratch_types=[
            pltpu.VMEM((_NCHUNK, _GCHUNK), jnp.int32),
            pltpu.VMEM((_BPW, _FEAT), jnp.float32),
            pltpu.VMEM((_BPW, _FEAT), jnp.float32),
            pltpu.VMEM((16,), jnp.float32),
            pltpu.SemaphoreType.DMA,
        ],
    )(_body)
    partial_losses = run(feats, lab, centers)
    return _ALPHA * (jnp.sum(partial_losses) / _BATCH)
